# baseline (speedup 1.0000x reference)
import functools

import jax
import jax.numpy as jnp
from jax import lax
from jax.experimental import pallas as pl
from jax.experimental.pallas import tpu as pltpu
from jax.experimental.pallas import tpu_sc as plsc

S, T = 4096, 200
D = 64
V = 1000000
NC, NS = 2, 16
NW = NC * NS
SPW = S // NW
BLK = 8
NBLK = SPW // BLK
NBUF = 2

_vector_mesh = plsc.VectorSubcoreMesh(
    core_axis_name="core", subcore_axis_name="subcore"
)


@jax.jit
def _gather_sc(wide, tok):
    @functools.partial(
        pl.kernel,
        out_type=jax.ShapeDtypeStruct((S, T, D), jnp.float32),
        mesh=_vector_mesh,
        scratch_types=[
            pltpu.VMEM((NBUF, BLK, T), jnp.int32),
            pltpu.VMEM((NBUF, 1, T, 2 * D), jnp.float32),
            pltpu.VMEM((NBUF, 1, T, D), jnp.float32),
            pltpu.SemaphoreType.DMA((NBUF,)),
            pltpu.SemaphoreType.DMA((NBUF,)),
            pltpu.SemaphoreType.DMA((NBUF,)),
        ],
    )
    def kern(wide_hbm, tok_hbm, out_hbm, idx_v, rows_v, sel_v, isem, gsem,
             osem):
        wid = lax.axis_index("subcore") * NC + lax.axis_index("core")
        base = wid * SPW

        for b in range(NBUF):
            pltpu.async_copy(
                tok_hbm.at[pl.ds(base + b * BLK, BLK)], idx_v.at[b],
                isem.at[b],
            )

        @pl.loop(0, NBLK, step=NBUF)
        def _(i):
            for b in range(NBUF):
                s0 = base + (i + b) * BLK

                pltpu.make_async_copy(
                    tok_hbm.at[pl.ds(s0, BLK)], idx_v.at[b], isem.at[b]
                ).wait()

                @pl.loop(0, BLK, step=NBUF)
                def _(jj):
                    for p in range(NBUF):
                        j = jj + p

                        if b == 0:
                            @pl.when((i + jj) > 0)
                            def _():
                                pltpu.make_async_copy(
                                    sel_v.at[p],
                                    out_hbm.at[pl.ds(s0, 1)],
                                    osem.at[p],
                                ).wait()
                        else:
                            pltpu.make_async_copy(
                                sel_v.at[p],
                                out_hbm.at[pl.ds(s0, 1)],
                                osem.at[p],
                            ).wait()

                        for g0, gn in ((0, 128), (128, T - 128)):
                            pltpu.async_copy(
                                wide_hbm.at[idx_v.at[b, j, pl.ds(g0, gn)]],
                                rows_v.at[p, 0, pl.ds(g0, gn)],
                                gsem.at[p],
                            )
                        for g0, gn in ((0, 128), (128, T - 128)):
                            pltpu.make_async_copy(
                                wide_hbm.at[idx_v.at[b, j, pl.ds(g0, gn)]],
                                rows_v.at[p, 0, pl.ds(g0, gn)],
                                gsem.at[p],
                            ).wait()

                        @pl.loop(0, T // 8)
                        def _(tt):
                            for l in range(8):
                                for k in range(4):
                                    sel_v[
                                        p, 0, tt * 8 + l,
                                        pl.ds(k * 16, 16),
                                    ] = rows_v[
                                        p, 0, tt * 8 + l,
                                        pl.ds(k * 16, 16),
                                    ]

                        pltpu.async_copy(
                            sel_v.at[p],
                            out_hbm.at[pl.ds(s0 + j, 1)],
                            osem.at[p],
                        )

                @pl.when(i + NBUF < NBLK)
                def _():
                    pltpu.async_copy(
                        tok_hbm.at[pl.ds(s0 + NBUF * BLK, BLK)],
                        idx_v.at[b],
                        isem.at[b],
                    )

        for p in range(NBUF):
            pltpu.make_async_copy(
                sel_v.at[p],
                out_hbm.at[pl.ds(base, 1)],
                osem.at[p],
            ).wait()

    return kern(wide, tok)


def kernel(tokenized_sentence, table):
    wide = jnp.concatenate([table, table], axis=1)
    return _gather_sc(wide, tokenized_sentence)


# speedup vs baseline: 1.0799x; 1.0799x over previous
import functools

import jax
import jax.numpy as jnp
from jax import lax
from jax.experimental import pallas as pl
from jax.experimental.pallas import tpu as pltpu
from jax.experimental.pallas import tpu_sc as plsc

S, T = 4096, 200
B = S * T
D = 64
NC, NS = 2, 16
NW = NC * NS
BPW = B // NW
C = 512
NCHUNK = BPW // C
NBUF = 2

_vector_mesh = plsc.VectorSubcoreMesh(
    core_axis_name="core", subcore_axis_name="subcore"
)


@jax.jit
def _gather_sc(table, indices):
    @functools.partial(
        pl.kernel,
        out_type=jax.ShapeDtypeStruct((B, D), jnp.float32),
        mesh=_vector_mesh,
        scratch_types=[
            pltpu.VMEM((NBUF, C), jnp.int32),
            pltpu.VMEM((NBUF, C, D), jnp.float32),
            pltpu.SemaphoreType.DMA((NBUF,)),
            pltpu.SemaphoreType.DMA((NBUF,)),
            pltpu.SemaphoreType.DMA((NBUF,)),
        ],
        compiler_params=pltpu.CompilerParams(use_tc_tiling_on_sc=False),
    )
    def kern(tab_hbm, idx_hbm, out_hbm, idx_v, rows_v, isem, gsem, osem):
        wid = lax.axis_index("subcore") * NC + lax.axis_index("core")
        base = wid * BPW

        for b in range(NBUF):
            pltpu.async_copy(
                idx_hbm.at[pl.ds(base + b * C, C)], idx_v.at[b], isem.at[b]
            )

        @pl.loop(0, NCHUNK, step=NBUF)
        def _(i):
            for b in range(NBUF):
                off = base + (i + b) * C

                @pl.when(i > 0)
                def _():
                    pltpu.make_async_copy(
                        rows_v.at[b], out_hbm.at[pl.ds(off, C)], osem.at[b]
                    ).wait()

                pltpu.make_async_copy(
                    idx_hbm.at[pl.ds(off, C)], idx_v.at[b], isem.at[b]
                ).wait()

                pltpu.async_copy(
                    tab_hbm.at[idx_v.at[b]], rows_v.at[b], gsem.at[b]
                ).wait()

                @pl.when(i + NBUF < NCHUNK)
                def _():
                    pltpu.async_copy(
                        idx_hbm.at[pl.ds(off + NBUF * C, C)],
                        idx_v.at[b],
                        isem.at[b],
                    )

                pltpu.async_copy(
                    rows_v.at[b], out_hbm.at[pl.ds(off, C)], osem.at[b]
                )

        for b in range(NBUF):
            pltpu.make_async_copy(
                rows_v.at[b], out_hbm.at[pl.ds(base, C)], osem.at[b]
            ).wait()

    return kern(table, indices)


def kernel(tokenized_sentence, table):
    idx = tokenized_sentence.reshape(B)
    out = _gather_sc(table, idx)
    return out.reshape(S, T, D)


# speedup vs baseline: 1.1777x; 1.0905x over previous
import functools

import jax
import jax.numpy as jnp
from jax import lax
from jax.experimental import pallas as pl
from jax.experimental.pallas import tpu as pltpu
from jax.experimental.pallas import tpu_sc as plsc

S, T = 4096, 200
D = 64
V = 1000000
NC, NS = 2, 16
NW = NC * NS
SPW = S // NW
BLK = 8
NBLK = SPW // BLK
NBUF = 2

_vector_mesh = plsc.VectorSubcoreMesh(
    core_axis_name="core", subcore_axis_name="subcore"
)


@jax.jit
def _gather_sc(wide, tok):
    @functools.partial(
        pl.kernel,
        out_type=jax.ShapeDtypeStruct((S, T, D), jnp.float32),
        mesh=_vector_mesh,
        scratch_types=[
            pltpu.VMEM((NBUF, BLK, T), jnp.int32),
            pltpu.VMEM((NBUF, 1, T, 2 * D), jnp.float32),
            pltpu.VMEM((NBUF, 1, T, D), jnp.float32),
            pltpu.SemaphoreType.DMA((NBUF,)),
            pltpu.SemaphoreType.DMA((NBUF,)),
            pltpu.SemaphoreType.DMA((NBUF,)),
        ],
    )
    def kern(wide_hbm, tok_hbm, out_hbm, idx_v, rows_v, sel_v, isem, gsem,
             osem):
        wid = lax.axis_index("subcore") * NC + lax.axis_index("core")
        base = wid * SPW

        for b in range(NBUF):
            pltpu.async_copy(
                tok_hbm.at[pl.ds(base + b * BLK, BLK)], idx_v.at[b],
                isem.at[b],
            )

        @pl.loop(0, NBLK, step=NBUF)
        def _(i):
            for b in range(NBUF):
                s0 = base + (i + b) * BLK

                pltpu.make_async_copy(
                    tok_hbm.at[pl.ds(s0, BLK)], idx_v.at[b], isem.at[b]
                ).wait()

                for g0, gn in ((0, 128), (128, T - 128)):
                    pltpu.async_copy(
                        wide_hbm.at[idx_v.at[b, 0, pl.ds(g0, gn)]],
                        rows_v.at[0, 0, pl.ds(g0, gn)],
                        gsem.at[0],
                    )

                for j in range(BLK):
                    p = j % NBUF

                    for g0, gn in ((0, 128), (128, T - 128)):
                        pltpu.make_async_copy(
                            wide_hbm.at[idx_v.at[b, j, pl.ds(g0, gn)]],
                            rows_v.at[p, 0, pl.ds(g0, gn)],
                            gsem.at[p],
                        ).wait()

                    if j + 1 < BLK:
                        for g0, gn in ((0, 128), (128, T - 128)):
                            pltpu.async_copy(
                                wide_hbm.at[
                                    idx_v.at[b, j + 1, pl.ds(g0, gn)]
                                ],
                                rows_v.at[1 - p, 0, pl.ds(g0, gn)],
                                gsem.at[1 - p],
                            )

                    if b == 0 and j < NBUF:
                        @pl.when(i > 0)
                        def _():
                            pltpu.make_async_copy(
                                sel_v.at[p],
                                out_hbm.at[pl.ds(s0, 1)],
                                osem.at[p],
                            ).wait()
                    else:
                        pltpu.make_async_copy(
                            sel_v.at[p],
                            out_hbm.at[pl.ds(s0, 1)],
                            osem.at[p],
                        ).wait()

                    @pl.loop(0, T // 8)
                    def _(tt):
                        for l in range(8):
                            for k in range(4):
                                sel_v[p, 0, tt * 8 + l, pl.ds(k * 16, 16)] = (
                                    rows_v[
                                        p, 0, tt * 8 + l, pl.ds(k * 16, 16)
                                    ]
                                )

                    pltpu.async_copy(
                        sel_v.at[p],
                        out_hbm.at[pl.ds(s0 + j, 1)],
                        osem.at[p],
                    )

                @pl.when(i + NBUF < NBLK)
                def _():
                    pltpu.async_copy(
                        tok_hbm.at[pl.ds(s0 + NBUF * BLK, BLK)],
                        idx_v.at[b],
                        isem.at[b],
                    )

        for p in range(NBUF):
            pltpu.make_async_copy(
                sel_v.at[p], out_hbm.at[pl.ds(base, 1)], osem.at[p]
            ).wait()

    return kern(wide, tok)


def kernel(tokenized_sentence, table):
    wide = jnp.pad(table, ((0, 0), (0, D)))
    return _gather_sc(wide, tokenized_sentence)
